# Initial kernel scaffold; baseline (speedup 1.0000x reference)
import jax
import jax.numpy as jnp
from jax.experimental import pallas as pl


def kernel(x, edge_index, batch, params):
    raise NotImplementedError("write your pallas kernel here")


# speedup vs baseline: 10.8251x; 10.8251x over previous
import functools

import jax
import jax.numpy as jnp
from jax import lax
from jax.experimental import pallas as pl
from jax.experimental.pallas import tpu as pltpu
from jax.experimental.pallas import tpu_sc as plsc

N_NODES = 10000
N_GRAPHS = 16
NODES_PER = 625
E_EDGES = 320000
D_FEAT = 128
HIDDEN = 16
LATENT = 8
N_LAYERS = 10

ADJ_WORDS = N_GRAPHS * NODES_PER * NODES_PER

N_TASKS = 64
CHUNK = 97656
LAST_CHUNK = ADJ_WORDS - 63 * CHUNK
ACC_WORDS = 97680
EDGE_BATCH = 8000
N_BATCHES = E_EDGES // EDGE_BATCH


def _sc_scatter_body(idx_hbm, out_hbm, acc, ebuf):
    core = lax.axis_index("c")
    sub = lax.axis_index("s")
    wid = sub * 2 + core

    for p in range(2):
        task = wid + 32 * p
        base = task * CHUNK
        is_last = task == N_TASKS - 1
        end = base + jnp.where(is_last, LAST_CHUNK, CHUNK)

        zeros16 = jnp.zeros((16,), jnp.float32)

        def _zero(i, _):
            acc[pl.ds(i * 16, 16)] = zeros16
            return 0

        lax.fori_loop(0, ACC_WORDS // 16, _zero, 0)

        def _batch(b, _):
            pltpu.sync_copy(idx_hbm.at[pl.ds(b * EDGE_BATCH, EDGE_BATCH)], ebuf)

            def _vec(j, _):
                v = ebuf[pl.ds(j * 16, 16)]
                inb = (v >= base) & (v < end)
                li = jnp.where(inb, v - base, 0)
                val = jnp.where(inb, 1.0, 0.0).astype(jnp.float32)
                plsc.addupdate_scatter(acc, [li], val)
                return 0

            lax.fori_loop(0, EDGE_BATCH // 16, _vec, 0)
            return 0

        lax.fori_loop(0, N_BATCHES, _batch, 0)

        @pl.when(is_last)
        def _():
            pltpu.sync_copy(acc.at[pl.ds(0, LAST_CHUNK)],
                            out_hbm.at[pl.ds(base, LAST_CHUNK)])

        @pl.when(jnp.logical_not(is_last))
        def _():
            pltpu.sync_copy(acc.at[pl.ds(0, CHUNK)],
                            out_hbm.at[pl.ds(base, CHUNK)])


def _build_dense_adj(flat_idx):
    mesh = plsc.VectorSubcoreMesh(core_axis_name="c", subcore_axis_name="s")
    fn = pl.kernel(
        _sc_scatter_body,
        out_type=jax.ShapeDtypeStruct((ADJ_WORDS,), jnp.float32),
        mesh=mesh,
        scratch_types=[
            pltpu.VMEM((ACC_WORDS,), jnp.float32),
            pltpu.VMEM((EDGE_BATCH,), jnp.int32),
        ],
        compiler_params=pltpu.CompilerParams(needs_layout_passes=False),
    )
    return fn(flat_idx)


def _tc_body(x_ref, adj_ref, eps_ref, w1a_ref, b1a_ref, w2a_ref, b2a_ref,
             wst1_ref, bst1_ref, wst2_ref, bst2_ref, wm_ref, bm_ref,
             ws_ref, bs_ref, pred_ref, nll_ref, kl_ref):
    g = pl.program_id(0)
    adj = adj_ref[0]
    h = x_ref[0]

    def mp(hh):
        return lax.dot_general(adj, hh, (((0,), (0,)), ((), ())),
                               preferred_element_type=jnp.float32,
                               precision=lax.Precision.HIGHEST)

    m = h + mp(h)
    m = jnp.maximum(jnp.dot(m, w1a_ref[...],
                            preferred_element_type=jnp.float32) + b1a_ref[...], 0.0)
    h = jnp.dot(m, w2a_ref[...], preferred_element_type=jnp.float32) + b2a_ref[...]
    h = jnp.maximum(h, 0.0)

    for i in range(N_LAYERS - 1):
        m = h + mp(h)
        m = jnp.maximum(jnp.dot(m, wst1_ref[i],
                                preferred_element_type=jnp.float32) + bst1_ref[i], 0.0)
        h = jnp.dot(m, wst2_ref[i], preferred_element_type=jnp.float32) + bst2_ref[i]
        if i < N_LAYERS - 2:
            h = jnp.maximum(h, 0.0)

    mu = jnp.dot(h, wm_ref[...], preferred_element_type=jnp.float32) + bm_ref[...]
    sx = jnp.dot(h, ws_ref[...], preferred_element_type=jnp.float32) + bs_ref[...]
    std = jnp.maximum(sx, 0.0) + jnp.log1p(jnp.exp(-jnp.abs(sx)))
    z = mu + std * eps_ref[0]

    logits = lax.dot_general(z, z, (((1,), (1,)), ((), ())),
                             preferred_element_type=jnp.float32)
    p = jax.nn.sigmoid(logits)
    pred_ref[0] = p

    pc = jnp.clip(p, 1e-7, 1.0 - 1e-7)
    nll = -jnp.sum(adj * jnp.log(pc) + (1.0 - adj) * jnp.log(1.0 - pc))
    kl = jnp.sum(-jnp.log(std) + 0.5 * (std * std + mu * mu) - 0.5)

    @pl.when(g == 0)
    def _():
        nll_ref[...] = jnp.zeros((1, 1), jnp.float32)
        kl_ref[...] = jnp.zeros((1, 1), jnp.float32)

    nll_ref[...] += jnp.reshape(nll, (1, 1))
    kl_ref[...] += jnp.reshape(kl, (1, 1))


def _run_tc(x3, adj3, eps3, weights):
    (w1a, b1a, w2a, b2a, wst1, bst1, wst2, bst2, wm, bm, ws, bs) = weights
    full = lambda a: pl.BlockSpec(a.shape, lambda g: (0,) * a.ndim)
    out_shapes = [
        jax.ShapeDtypeStruct((N_GRAPHS, NODES_PER, NODES_PER), jnp.float32),
        jax.ShapeDtypeStruct((1, 1), jnp.float32),
        jax.ShapeDtypeStruct((1, 1), jnp.float32),
    ]
    return pl.pallas_call(
        _tc_body,
        grid=(N_GRAPHS,),
        in_specs=[
            pl.BlockSpec((1, NODES_PER, D_FEAT), lambda g: (g, 0, 0)),
            pl.BlockSpec((1, NODES_PER, NODES_PER), lambda g: (g, 0, 0)),
            pl.BlockSpec((1, NODES_PER, LATENT), lambda g: (g, 0, 0)),
            full(w1a), full(b1a), full(w2a), full(b2a),
            full(wst1), full(bst1), full(wst2), full(bst2),
            full(wm), full(bm), full(ws), full(bs),
        ],
        out_specs=[
            pl.BlockSpec((1, NODES_PER, NODES_PER), lambda g: (g, 0, 0)),
            pl.BlockSpec((1, 1), lambda g: (0, 0)),
            pl.BlockSpec((1, 1), lambda g: (0, 0)),
        ],
        out_shape=out_shapes,
        compiler_params=pltpu.CompilerParams(
            dimension_semantics=("arbitrary",)),
    )(x3, adj3, eps3, w1a, b1a, w2a, b2a, wst1, bst1, wst2, bst2,
      wm, bm, ws, bs)


def kernel(x, edge_index, batch, params):
    src = edge_index[0]
    dst = edge_index[1]
    gsrc = src // NODES_PER
    flat_idx = (gsrc * NODES_PER + (src - gsrc * NODES_PER)) * NODES_PER \
        + (dst % NODES_PER)

    dense_flat = _build_dense_adj(flat_idx.astype(jnp.int32))
    dense_adj = dense_flat.reshape(N_GRAPHS, NODES_PER, NODES_PER)

    eps = jax.random.normal(jax.random.key(42), (N_NODES, LATENT),
                            dtype=jnp.float32)
    x3 = x.reshape(N_GRAPHS, NODES_PER, D_FEAT)
    eps3 = eps.reshape(N_GRAPHS, NODES_PER, LATENT)

    p0 = params['gin0']
    w1a, b1a = p0['W1'], p0['b1'].reshape(1, HIDDEN)
    w2a, b2a = p0['W2'], p0['b2'].reshape(1, HIDDEN)
    wst1 = jnp.stack([params['gin%d' % i]['W1'] for i in range(1, N_LAYERS)])
    bst1 = jnp.stack([params['gin%d' % i]['b1'].reshape(1, HIDDEN)
                      for i in range(1, N_LAYERS)])
    wst2 = jnp.stack([params['gin%d' % i]['W2'] for i in range(1, N_LAYERS)])
    bst2 = jnp.stack([params['gin%d' % i]['b2'].reshape(1, HIDDEN)
                      for i in range(1, N_LAYERS)])
    wm, bm = params['Wm'], params['bm'].reshape(1, LATENT)
    ws, bs = params['Ws'], params['bs'].reshape(1, LATENT)

    adj_pred, nll, kl = _run_tc(
        x3, dense_adj, eps3,
        (w1a, b1a, w2a, b2a, wst1, bst1, wst2, bst2, wm, bm, ws, bs))

    return nll[0, 0], kl[0, 0], adj_pred, dense_adj
